# baseline (speedup 1.0000x reference)
import functools

import jax
import jax.numpy as jnp
from jax import lax
from jax.experimental import pallas as pl
from jax.experimental.pallas import tpu as pltpu
from jax.experimental.pallas import tpu_sc as plsc


def _scores_body(x_ref, wt_ref, b_ref, vw_ref, vb_ref, s_ref):
    xb = x_ref[...]
    h = jnp.tanh(
        lax.dot_general(xb, wt_ref[...], (((1,), (0,)), ((), ())),
                        preferred_element_type=jnp.float32,
                        precision=lax.Precision.DEFAULT)
        + b_ref[...])
    s = lax.dot_general(h, vw_ref[...], (((1,), (0,)), ((), ())),
                        preferred_element_type=jnp.float32,
                        precision=lax.Precision.DEFAULT)
    s_ref[...] = s + vb_ref[0, 0]


def _sc_mask(scores, k_rank):
    nb, nt = scores.shape
    npr = nt // 16
    nchunk = npr // 16
    kk = jnp.int32(k_rank)
    MIN32 = jnp.int32(-2**31)

    def body(s_hbm, maw_hbm, s_v, k_v, o_v, bf_f, bf_i, all_f, all_i, h2_v,
             sh_f, sh_i):
        cid = lax.axis_index("c")
        wid = lax.axis_index("s")
        iota = lax.iota(jnp.int32, 16)

        def xmax_f(v):
            for st in (8, 4, 2, 1):
                bf_f[...] = v
                v = jnp.maximum(v, plsc.load_gather(bf_f, [iota ^ st]))
            return v

        def xsum_f(v):
            for st in (8, 4, 2, 1):
                bf_f[...] = v
                v = v + plsc.load_gather(bf_f, [iota ^ st])
            return v

        def prefix_i(v):
            for st in (1, 2, 4, 8):
                bf_i[...] = v
                g = plsc.load_gather(bf_i, [jnp.maximum(iota - st, 0)])
                v = v + jnp.where(iota >= st, g, jnp.int32(0))
            return v

        def xchg_f(vec, buf):
            bf_f[...] = vec
            pltpu.sync_copy(bf_f, sh_f.at[pl.ds(buf * 256 + wid * 16, 16)])
            plsc.subcore_barrier()
            pltpu.sync_copy(sh_f.at[pl.ds(buf * 256, 256)], all_f)
            return plsc.load_gather(all_f, [iota * 16])

        def xchg_i(vec, buf):
            bf_i[...] = vec
            pltpu.sync_copy(bf_i, sh_i.at[pl.ds(buf * 256 + wid * 16, 16)])
            plsc.subcore_barrier()
            pltpu.sync_copy(sh_i.at[pl.ds(buf * 256, 256)], all_i)
            return plsc.load_gather(all_i, [iota * 16])

        def row_body(rr, carry):
            row = cid * (nb // 2) + rr
            base = row * nt + wid * npr
            pltpu.sync_copy(s_hbm.at[pl.ds(base, npr)], s_v)

            def sw1(i, mx):
                s = s_v[pl.ds(i * 16, 16)]
                s = jnp.where(s == 0.0, 0.0, s)
                bits = lax.bitcast_convert_type(s, jnp.int32)
                k_v[pl.ds(i * 16, 16)] = bits ^ ((bits >> 31) | MIN32)
                return jnp.maximum(mx, s)

            mxv = lax.fori_loop(
                0, nchunk, sw1, jnp.full((16,), -jnp.inf, jnp.float32))
            gmax = xmax_f(xchg_f(xmax_f(mxv), 0))

            def sw2(i, acc):
                e = jnp.exp(s_v[pl.ds(i * 16, 16)] - gmax)
                o_v[pl.ds(i * 16, 16)] = e
                return acc + e

            accv = lax.fori_loop(
                0, nchunk, sw2, jnp.zeros((16,), jnp.float32))
            z = xsum_f(xchg_f(xsum_f(accv), 1))
            inv_z = 1.0 / z

            zi = jnp.zeros((16,), jnp.int32)
            ones = jnp.full((16,), 1, jnp.int32)

            def rnd(t, st):
                p_u, hi_mask, rcur = st
                shift = zi + (jnp.int32(28) - 4 * t)

                def zb(l, c):
                    h2_v[pl.ds(l * 16, 16)] = zi
                    return c
                lax.fori_loop(0, 16, zb, 0)

                def hsweep(i, c):
                    ku = k_v[pl.ds(i * 16, 16)]
                    nib = lax.shift_right_logical(ku, shift) & jnp.int32(0xF)
                    match = (ku & hi_mask) == p_u
                    plsc.addupdate_scatter(
                        h2_v, [nib * 16 + iota], ones, mask=match)
                    return c
                lax.fori_loop(0, nchunk, hsweep, 0)

                lbin = zi
                for l in range(16):
                    lbin = lbin + plsc.load_gather(h2_v, [iota * 16 + l])

                bf_i[...] = lbin
                pltpu.sync_copy(
                    bf_i, sh_i.at[pl.ds((t % 2) * 256 + wid * 16, 16)])
                plsc.subcore_barrier()
                pltpu.sync_copy(sh_i.at[pl.ds((t % 2) * 256, 256)], all_i)
                gbin = zi
                for l in range(16):
                    gbin = gbin + all_i[pl.ds(l * 16, 16)]

                cum = prefix_i(gbin)
                chosen = plsc.all_reduce_population_count(cum < rcur)
                bf_i[...] = cum - gbin
                sel_excl = plsc.load_gather(bf_i, [chosen])
                p_u = p_u | lax.shift_left(chosen, shift)
                hi_mask = hi_mask | lax.shift_left(zi + 0xF, shift)
                return (p_u, hi_mask, rcur - sel_excl)

            theta, _, r = lax.fori_loop(0, 8, rnd, (zi, zi, zi + kk))

            def tcnt(i, acc):
                kv = k_v[pl.ds(i * 16, 16)]
                return acc + plsc.all_reduce_population_count(kv == theta)

            ltie = lax.fori_loop(0, nchunk, tcnt, zi)
            pt = xchg_i(ltie, 0)
            excl = prefix_i(pt) - pt
            bf_i[...] = excl
            before = plsc.load_gather(
                bf_i, [jnp.full((16,), 0, jnp.int32) + wid])
            rem = r - before

            theta_s = theta ^ MIN32

            def sw3(i, run):
                kv = k_v[pl.ds(i * 16, 16)]
                tie = kv == theta
                pp = prefix_i(jnp.where(tie, jnp.int32(1), jnp.int32(0)))
                sel = tie & ((run + pp) <= rem)
                masked = ((kv ^ MIN32) < theta_s) | sel
                o_v[pl.ds(i * 16, 16)] = jnp.where(
                    masked, 0.0, o_v[pl.ds(i * 16, 16)] * inv_z)
                return run + plsc.all_reduce_population_count(tie)

            lax.fori_loop(0, nchunk, sw3, zi)
            pltpu.sync_copy(o_v, maw_hbm.at[pl.ds(base, npr)])
            return carry

        lax.fori_loop(0, nb // 2, row_body, jnp.int32(0))

    return pl.kernel(
        body,
        out_type=jax.ShapeDtypeStruct((nb * nt,), jnp.float32),
        mesh=plsc.VectorSubcoreMesh(core_axis_name="c", subcore_axis_name="s"),
        compiler_params=pltpu.CompilerParams(needs_layout_passes=False),
        scratch_types=[
            pltpu.VMEM((npr,), jnp.float32),
            pltpu.VMEM((npr,), jnp.int32),
            pltpu.VMEM((npr,), jnp.float32),
            pltpu.VMEM((16,), jnp.float32),
            pltpu.VMEM((16,), jnp.int32),
            pltpu.VMEM((256,), jnp.float32),
            pltpu.VMEM((256,), jnp.int32),
            pltpu.VMEM((256,), jnp.int32),
            pltpu.VMEM_SHARED((512,), jnp.float32),
            pltpu.VMEM_SHARED((512,), jnp.int32),
        ],
    )(scores.reshape(nb * nt))


def _mask_body(s_ref, maw_ref, *, k_rank):
    s = s_ref[...]
    s = jnp.where(s == 0.0, 0.0, s)
    bits = lax.bitcast_convert_type(s, jnp.int32)
    key = bits ^ ((bits >> 31) & jnp.int32(0x7FFFFFFF))
    nb, nt = s.shape

    mx = jnp.max(s, axis=1, keepdims=True)
    z = jnp.sum(jnp.exp(s - mx), axis=1, keepdims=True)

    MIN32 = jnp.int32(-2**31)
    kk = jnp.int32(k_rank)

    def bs_body(i, carry):
        p_u, c_less = carry
        bit = jnp.int32(1) << (jnp.int32(31) - i)
        q_u = p_u | bit
        thr = q_u ^ MIN32
        c = jnp.sum((key < thr).astype(jnp.int32), axis=1, keepdims=True)
        accept = c < kk
        return (jnp.where(accept, q_u, p_u), jnp.where(accept, c, c_less))

    p0 = jnp.zeros((nb, 1), jnp.int32)
    p_u, c_less = lax.fori_loop(0, 32, bs_body, (p0, p0))
    theta = p_u ^ MIN32

    is_tie = key == theta
    r = kk - c_less
    idx = lax.broadcasted_iota(jnp.int32, s.shape, 1)

    def bs2_body(i, p2):
        bit = jnp.int32(1) << (jnp.int32(12) - i)
        q = p2 | bit
        c = jnp.sum((is_tie & (idx < q)).astype(jnp.int32), axis=1, keepdims=True)
        return jnp.where(c < r, q, p2)

    cut = lax.fori_loop(0, 13, bs2_body, jnp.zeros((nb, 1), jnp.int32))

    masked = (key < theta) | (is_tie & (idx <= cut))
    maw_ref[...] = jnp.where(masked, 0.0, jnp.exp(s - mx) / z)


def _apply_body(x_ref, m_ref, h0_ref, o_ref):
    maw = m_ref[...]
    o_ref[...] = x_ref[...] * maw + (1.0 - maw) * h0_ref[...]


def kernel(x, W, b, v_w, v_b, H0):
    nb, nt, nd = x.shape
    k_rank = int(nt * 0.7)
    x2 = x.reshape(nb * nt, nd)
    wt = W.T

    tba = 2048
    nh = nb // 2
    nblk = nh * nt // tba

    def scores_half(h):
        return pl.pallas_call(
            _scores_body,
            grid=(nblk,),
            in_specs=[
                pl.BlockSpec((tba, nd), lambda i, h=h: (i + h * nblk, 0)),
                pl.BlockSpec((nd, nd), lambda i: (0, 0)),
                pl.BlockSpec((1, nd), lambda i: (0, 0)),
                pl.BlockSpec((nd, 1), lambda i: (0, 0)),
                pl.BlockSpec((1, 1), lambda i: (0, 0)),
            ],
            out_specs=pl.BlockSpec((tba, 1), lambda i: (i, 0)),
            out_shape=jax.ShapeDtypeStruct((nh * nt, 1), jnp.float32),
            compiler_params=pltpu.CompilerParams(
                dimension_semantics=("arbitrary",)),
        )(x2, wt, b.reshape(1, nd), v_w.reshape(nd, 1), v_b.reshape(1, 1))

    maw_halves = [
        _sc_mask(scores_half(h).reshape(nh, nt), k_rank) for h in range(2)]
    maw = jnp.concatenate(maw_halves).reshape(nb, nt)

    tbc = 2048

    def apply_half(h):
        return pl.pallas_call(
            _apply_body,
            grid=(nh * nt // tbc,),
            in_specs=[
                pl.BlockSpec((tbc, nd), lambda i: (i, 0)),
                pl.BlockSpec((tbc, 1), lambda i: (i, 0)),
                pl.BlockSpec((1, nd), lambda i: (0, 0)),
            ],
            out_specs=pl.BlockSpec((tbc, nd), lambda i: (i, 0)),
            out_shape=jax.ShapeDtypeStruct((nh * nt, nd), jnp.float32),
            compiler_params=pltpu.CompilerParams(
                dimension_semantics=("parallel",)),
        )(x2[h * nh * nt:(h + 1) * nh * nt],
          maw_halves[h].reshape(nh * nt, 1), H0.reshape(1, nd))

    out = jnp.concatenate([apply_half(0), apply_half(1)])
    return out.reshape(nb, nt, nd), maw


# speedup vs baseline: 1.6594x; 1.6594x over previous
import functools

import jax
import jax.numpy as jnp
from jax import lax
from jax.experimental import pallas as pl
from jax.experimental.pallas import tpu as pltpu
from jax.experimental.pallas import tpu_sc as plsc


def _scores_body(x_ref, wt_ref, b_ref, vw_ref, vb_ref, s_ref):
    xb = x_ref[...]
    h = jnp.tanh(
        lax.dot_general(xb, wt_ref[...], (((1,), (0,)), ((), ())),
                        preferred_element_type=jnp.float32,
                        precision=lax.Precision.DEFAULT)
        + b_ref[...])
    s = lax.dot_general(h, vw_ref[...], (((1,), (0,)), ((), ())),
                        preferred_element_type=jnp.float32,
                        precision=lax.Precision.DEFAULT)
    s_ref[...] = s + vb_ref[0, 0]


def _sc_mask(scores, k_rank):
    nb, nt = scores.shape
    npr = nt // 16
    nchunk = npr // 16
    kk = jnp.int32(k_rank)
    MIN32 = jnp.int32(-2**31)

    def body(s_hbm, maw_hbm, s_v, k_v, o_v, bf_f, bf_i, all_f, all_i, h2_v,
             sh_f, sh_i):
        cid = lax.axis_index("c")
        wid = lax.axis_index("s")
        iota = lax.iota(jnp.int32, 16)

        def xmax_f(v):
            for st in (8, 4, 2, 1):
                bf_f[...] = v
                v = jnp.maximum(v, plsc.load_gather(bf_f, [iota ^ st]))
            return v

        def xsum_f(v):
            for st in (8, 4, 2, 1):
                bf_f[...] = v
                v = v + plsc.load_gather(bf_f, [iota ^ st])
            return v

        def prefix_i(v):
            for st in (1, 2, 4, 8):
                bf_i[...] = v
                g = plsc.load_gather(bf_i, [jnp.maximum(iota - st, 0)])
                v = v + jnp.where(iota >= st, g, jnp.int32(0))
            return v

        def xchg_f(vec, buf):
            bf_f[...] = vec
            pltpu.sync_copy(bf_f, sh_f.at[pl.ds(buf * 256 + wid * 16, 16)])
            plsc.subcore_barrier()
            pltpu.sync_copy(sh_f.at[pl.ds(buf * 256, 256)], all_f)
            return plsc.load_gather(all_f, [iota * 16])

        def xchg_i(vec, buf):
            bf_i[...] = vec
            pltpu.sync_copy(bf_i, sh_i.at[pl.ds(buf * 256 + wid * 16, 16)])
            plsc.subcore_barrier()
            pltpu.sync_copy(sh_i.at[pl.ds(buf * 256, 256)], all_i)
            return plsc.load_gather(all_i, [iota * 16])

        def row_body(rr, carry):
            row = cid * (nb // 2) + rr
            base = row * nt + wid * npr
            pltpu.sync_copy(s_hbm.at[pl.ds(base, npr)], s_v)

            def sw1(i, mx):
                s = s_v[pl.ds(i * 16, 16)]
                s = jnp.where(s == 0.0, 0.0, s)
                bits = lax.bitcast_convert_type(s, jnp.int32)
                k_v[pl.ds(i * 16, 16)] = bits ^ ((bits >> 31) | MIN32)
                return jnp.maximum(mx, s)

            mxv = lax.fori_loop(
                0, nchunk, sw1, jnp.full((16,), -jnp.inf, jnp.float32))
            gmax = xmax_f(xchg_f(xmax_f(mxv), 0))

            def sw2(i, acc):
                e = jnp.exp(s_v[pl.ds(i * 16, 16)] - gmax)
                o_v[pl.ds(i * 16, 16)] = e
                return acc + e

            accv = lax.fori_loop(
                0, nchunk, sw2, jnp.zeros((16,), jnp.float32))
            z = xsum_f(xchg_f(xsum_f(accv), 1))
            inv_z = 1.0 / z

            zi = jnp.zeros((16,), jnp.int32)
            ones = jnp.full((16,), 1, jnp.int32)

            def rnd(t, st):
                p_u, hi_mask, rcur = st
                shift = zi + (jnp.int32(28) - 4 * t)

                def zb(l, c):
                    h2_v[pl.ds(l * 16, 16)] = zi
                    return c
                lax.fori_loop(0, 16, zb, 0)

                def hsweep(i, c):
                    ku = k_v[pl.ds(i * 16, 16)]
                    nib = lax.shift_right_logical(ku, shift) & jnp.int32(0xF)
                    match = (ku & hi_mask) == p_u
                    plsc.addupdate_scatter(
                        h2_v, [nib * 16 + iota], ones, mask=match)
                    return c
                lax.fori_loop(0, nchunk, hsweep, 0)

                lbin = zi
                for l in range(16):
                    lbin = lbin + plsc.load_gather(h2_v, [iota * 16 + l])

                bf_i[...] = lbin
                pltpu.sync_copy(
                    bf_i, sh_i.at[pl.ds((t % 2) * 256 + wid * 16, 16)])
                plsc.subcore_barrier()
                pltpu.sync_copy(sh_i.at[pl.ds((t % 2) * 256, 256)], all_i)
                gbin = zi
                for l in range(16):
                    gbin = gbin + all_i[pl.ds(l * 16, 16)]

                cum = prefix_i(gbin)
                chosen = plsc.all_reduce_population_count(cum < rcur)
                bf_i[...] = cum - gbin
                sel_excl = plsc.load_gather(bf_i, [chosen])
                p_u = p_u | lax.shift_left(chosen, shift)
                hi_mask = hi_mask | lax.shift_left(zi + 0xF, shift)
                return (p_u, hi_mask, rcur - sel_excl)

            theta, _, r = lax.fori_loop(0, 8, rnd, (zi, zi, zi + kk))

            def tcnt(i, acc):
                kv = k_v[pl.ds(i * 16, 16)]
                return acc + plsc.all_reduce_population_count(kv == theta)

            ltie = lax.fori_loop(0, nchunk, tcnt, zi)
            pt = xchg_i(ltie, 0)
            excl = prefix_i(pt) - pt
            bf_i[...] = excl
            before = plsc.load_gather(
                bf_i, [jnp.full((16,), 0, jnp.int32) + wid])
            rem = r - before

            theta_s = theta ^ MIN32

            def sw3(i, run):
                kv = k_v[pl.ds(i * 16, 16)]
                tie = kv == theta
                pp = prefix_i(jnp.where(tie, jnp.int32(1), jnp.int32(0)))
                sel = tie & ((run + pp) <= rem)
                masked = ((kv ^ MIN32) < theta_s) | sel
                o_v[pl.ds(i * 16, 16)] = jnp.where(
                    masked, 0.0, o_v[pl.ds(i * 16, 16)] * inv_z)
                return run + plsc.all_reduce_population_count(tie)

            lax.fori_loop(0, nchunk, sw3, zi)
            pltpu.sync_copy(o_v, maw_hbm.at[pl.ds(base, npr)])
            return carry

        lax.fori_loop(0, nb // 2, row_body, jnp.int32(0))

    return pl.kernel(
        body,
        out_type=jax.ShapeDtypeStruct((nb * nt,), jnp.float32),
        mesh=plsc.VectorSubcoreMesh(core_axis_name="c", subcore_axis_name="s"),
        compiler_params=pltpu.CompilerParams(needs_layout_passes=False),
        scratch_types=[
            pltpu.VMEM((npr,), jnp.float32),
            pltpu.VMEM((npr,), jnp.int32),
            pltpu.VMEM((npr,), jnp.float32),
            pltpu.VMEM((16,), jnp.float32),
            pltpu.VMEM((16,), jnp.int32),
            pltpu.VMEM((256,), jnp.float32),
            pltpu.VMEM((256,), jnp.int32),
            pltpu.VMEM((256,), jnp.int32),
            pltpu.VMEM_SHARED((512,), jnp.float32),
            pltpu.VMEM_SHARED((512,), jnp.int32),
        ],
    )(scores.reshape(nb * nt))


def _mask_body(s_ref, maw_ref, *, k_rank):
    s = s_ref[...]
    s = jnp.where(s == 0.0, 0.0, s)
    bits = lax.bitcast_convert_type(s, jnp.int32)
    key = bits ^ ((bits >> 31) & jnp.int32(0x7FFFFFFF))
    nb, nt = s.shape

    mx = jnp.max(s, axis=1, keepdims=True)
    z = jnp.sum(jnp.exp(s - mx), axis=1, keepdims=True)

    MIN32 = jnp.int32(-2**31)
    kk = jnp.int32(k_rank)

    def bs_body(i, carry):
        p_u, c_less = carry
        bit = jnp.int32(1) << (jnp.int32(31) - i)
        q_u = p_u | bit
        thr = q_u ^ MIN32
        c = jnp.sum((key < thr).astype(jnp.int32), axis=1, keepdims=True)
        accept = c < kk
        return (jnp.where(accept, q_u, p_u), jnp.where(accept, c, c_less))

    p0 = jnp.zeros((nb, 1), jnp.int32)
    p_u, c_less = lax.fori_loop(0, 32, bs_body, (p0, p0))
    theta = p_u ^ MIN32

    is_tie = key == theta
    r = kk - c_less
    idx = lax.broadcasted_iota(jnp.int32, s.shape, 1)

    def bs2_body(i, p2):
        bit = jnp.int32(1) << (jnp.int32(12) - i)
        q = p2 | bit
        c = jnp.sum((is_tie & (idx < q)).astype(jnp.int32), axis=1, keepdims=True)
        return jnp.where(c < r, q, p2)

    cut = lax.fori_loop(0, 13, bs2_body, jnp.zeros((nb, 1), jnp.int32))

    masked = (key < theta) | (is_tie & (idx <= cut))
    maw_ref[...] = jnp.where(masked, 0.0, jnp.exp(s - mx) / z)


def _apply_body(x_ref, m_ref, h0_ref, o_ref):
    maw = m_ref[...]
    o_ref[...] = x_ref[...] * maw + (1.0 - maw) * h0_ref[...]


def kernel(x, W, b, v_w, v_b, H0):
    nb, nt, nd = x.shape
    k_rank = int(nt * 0.7)
    x2 = x.reshape(nb * nt, nd)
    wt = W.T

    tba = 2048
    nblk = nb * nt // tba

    scores = pl.pallas_call(
        _scores_body,
        grid=(nblk,),
        in_specs=[
            pl.BlockSpec((tba, nd), lambda i: (i, 0)),
            pl.BlockSpec((nd, nd), lambda i: (0, 0)),
            pl.BlockSpec((1, nd), lambda i: (0, 0)),
            pl.BlockSpec((nd, 1), lambda i: (0, 0)),
            pl.BlockSpec((1, 1), lambda i: (0, 0)),
        ],
        out_specs=pl.BlockSpec((tba, 1), lambda i: (i, 0)),
        out_shape=jax.ShapeDtypeStruct((nb * nt, 1), jnp.float32),
        compiler_params=pltpu.CompilerParams(
            dimension_semantics=("arbitrary",)),
    )(x2, wt, b.reshape(1, nd), v_w.reshape(nd, 1), v_b.reshape(1, 1))

    maw = _sc_mask(scores.reshape(nb, nt), k_rank).reshape(nb, nt)

    tbc = 2048
    out = pl.pallas_call(
        _apply_body,
        grid=(nb * nt // tbc,),
        in_specs=[
            pl.BlockSpec((tbc, nd), lambda i: (i, 0)),
            pl.BlockSpec((tbc, 1), lambda i: (i, 0)),
            pl.BlockSpec((1, nd), lambda i: (0, 0)),
        ],
        out_specs=pl.BlockSpec((tbc, nd), lambda i: (i, 0)),
        out_shape=jax.ShapeDtypeStruct((nb * nt, nd), jnp.float32),
        compiler_params=pltpu.CompilerParams(
            dimension_semantics=("parallel",)),
    )(x2, maw.reshape(nb * nt, 1), H0.reshape(1, nd))

    return out.reshape(nb, nt, nd), maw


# speedup vs baseline: 1.6636x; 1.0025x over previous
import functools

import jax
import jax.numpy as jnp
from jax import lax
from jax.experimental import pallas as pl
from jax.experimental.pallas import tpu as pltpu
from jax.experimental.pallas import tpu_sc as plsc


def _scores_body(x_ref, wt_ref, b_ref, vw_ref, vb_ref, s_ref):
    xb = x_ref[...]
    h = jnp.tanh(
        lax.dot_general(xb, wt_ref[...], (((1,), (0,)), ((), ())),
                        preferred_element_type=jnp.float32,
                        precision=lax.Precision.DEFAULT)
        + b_ref[...])
    s = lax.dot_general(h, vw_ref[...], (((1,), (0,)), ((), ())),
                        preferred_element_type=jnp.float32,
                        precision=lax.Precision.DEFAULT)
    s_ref[...] = s + vb_ref[0, 0]


def _sc_mask(scores, k_rank):
    nb, nt = scores.shape
    npr = nt // 16
    nchunk = npr // 16
    kk = jnp.int32(k_rank)
    MIN32 = jnp.int32(-2**31)

    def body(s_hbm, maw_hbm, s_v, k_v, o_v, bf_f, bf_i, all_f, all_i, h2_v,
             sh_f, sh_i):
        cid = lax.axis_index("c")
        wid = lax.axis_index("s")
        iota = lax.iota(jnp.int32, 16)

        def xmax_f(v):
            for st in (8, 4, 2, 1):
                bf_f[...] = v
                v = jnp.maximum(v, plsc.load_gather(bf_f, [iota ^ st]))
            return v

        def xsum_f(v):
            for st in (8, 4, 2, 1):
                bf_f[...] = v
                v = v + plsc.load_gather(bf_f, [iota ^ st])
            return v

        def prefix_i(v):
            for st in (1, 2, 4, 8):
                bf_i[...] = v
                g = plsc.load_gather(bf_i, [jnp.maximum(iota - st, 0)])
                v = v + jnp.where(iota >= st, g, jnp.int32(0))
            return v

        def xchg_f(vec, buf):
            bf_f[...] = vec
            pltpu.sync_copy(bf_f, sh_f.at[pl.ds(buf * 256 + wid * 16, 16)])
            plsc.subcore_barrier()
            pltpu.sync_copy(sh_f.at[pl.ds(buf * 256, 256)], all_f)
            return plsc.load_gather(all_f, [iota * 16])

        def row_body(rr, carry):
            row = cid * (nb // 2) + rr
            base = row * nt + wid * npr
            pltpu.sync_copy(s_hbm.at[pl.ds(base, npr)], s_v)

            def sw1(i, mx):
                s = s_v[pl.ds(i * 16, 16)]
                s = jnp.where(s == 0.0, 0.0, s)
                bits = lax.bitcast_convert_type(s, jnp.int32)
                k_v[pl.ds(i * 16, 16)] = bits ^ ((bits >> 31) | MIN32)
                return jnp.maximum(mx, s)

            mxv = lax.fori_loop(
                0, nchunk, sw1, jnp.full((16,), -jnp.inf, jnp.float32))
            gmax = xmax_f(xchg_f(xmax_f(mxv), 0))

            def sw2(i, acc):
                e = jnp.exp(s_v[pl.ds(i * 16, 16)] - gmax)
                o_v[pl.ds(i * 16, 16)] = e
                return acc + e

            accv = lax.fori_loop(
                0, nchunk, sw2, jnp.zeros((16,), jnp.float32))
            z = xsum_f(xchg_f(xsum_f(accv), 1))
            inv_z = 1.0 / z

            zi = jnp.zeros((16,), jnp.int32)
            ones = jnp.full((16,), 1, jnp.int32)

            def rnd(t, st, final=False):
                p_u, hi_mask, rcur, before = st
                shift = zi + (jnp.int32(28) - 4 * t)

                def zb(l, c):
                    h2_v[pl.ds(l * 16, 16)] = zi
                    return c
                lax.fori_loop(0, 16, zb, 0)

                def hsweep(i, c):
                    ku = k_v[pl.ds(i * 16, 16)]
                    nib = lax.shift_right_logical(ku, shift) & jnp.int32(0xF)
                    match = (ku & hi_mask) == p_u
                    plsc.addupdate_scatter(
                        h2_v, [nib * 16 + iota], ones, mask=match)
                    return c
                lax.fori_loop(0, nchunk, hsweep, 0)

                lbin = zi
                for l in range(16):
                    lbin = lbin + plsc.load_gather(h2_v, [iota * 16 + l])

                bf_i[...] = lbin
                pltpu.sync_copy(
                    bf_i, sh_i.at[pl.ds((t % 2) * 256 + wid * 16, 16)])
                plsc.subcore_barrier()
                pltpu.sync_copy(sh_i.at[pl.ds((t % 2) * 256, 256)], all_i)
                gbin = zi
                for l in range(16):
                    gbin = gbin + all_i[pl.ds(l * 16, 16)]

                cum = prefix_i(gbin)
                chosen = plsc.all_reduce_population_count(cum < rcur)
                bf_i[...] = cum - gbin
                sel_excl = plsc.load_gather(bf_i, [chosen])
                p_u = p_u | lax.shift_left(chosen, shift)
                hi_mask = hi_mask | lax.shift_left(zi + 0xF, shift)
                if final:
                    pcnt = plsc.load_gather(all_i, [iota * 16 + chosen])
                    excl_v = prefix_i(pcnt) - pcnt
                    bf_i[...] = excl_v
                    before = plsc.load_gather(bf_i, [zi + wid])
                return (p_u, hi_mask, rcur - sel_excl, before)

            st7 = lax.fori_loop(0, 7, rnd, (zi, zi, zi + kk, zi))
            theta, _, r, before = rnd(7, st7, final=True)
            rem = r - before

            theta_s = theta ^ MIN32

            def sw3(i, run):
                kv = k_v[pl.ds(i * 16, 16)]
                tie = kv == theta
                pp = prefix_i(jnp.where(tie, jnp.int32(1), jnp.int32(0)))
                sel = tie & ((run + pp) <= rem)
                masked = ((kv ^ MIN32) < theta_s) | sel
                o_v[pl.ds(i * 16, 16)] = jnp.where(
                    masked, 0.0, o_v[pl.ds(i * 16, 16)] * inv_z)
                return run + plsc.all_reduce_population_count(tie)

            lax.fori_loop(0, nchunk, sw3, zi)
            pltpu.sync_copy(o_v, maw_hbm.at[pl.ds(base, npr)])
            return carry

        lax.fori_loop(0, nb // 2, row_body, jnp.int32(0))

    return pl.kernel(
        body,
        out_type=jax.ShapeDtypeStruct((nb * nt,), jnp.float32),
        mesh=plsc.VectorSubcoreMesh(core_axis_name="c", subcore_axis_name="s"),
        compiler_params=pltpu.CompilerParams(needs_layout_passes=False),
        scratch_types=[
            pltpu.VMEM((npr,), jnp.float32),
            pltpu.VMEM((npr,), jnp.int32),
            pltpu.VMEM((npr,), jnp.float32),
            pltpu.VMEM((16,), jnp.float32),
            pltpu.VMEM((16,), jnp.int32),
            pltpu.VMEM((256,), jnp.float32),
            pltpu.VMEM((256,), jnp.int32),
            pltpu.VMEM((256,), jnp.int32),
            pltpu.VMEM_SHARED((512,), jnp.float32),
            pltpu.VMEM_SHARED((512,), jnp.int32),
        ],
    )(scores.reshape(nb * nt))


def _mask_body(s_ref, maw_ref, *, k_rank):
    s = s_ref[...]
    s = jnp.where(s == 0.0, 0.0, s)
    bits = lax.bitcast_convert_type(s, jnp.int32)
    key = bits ^ ((bits >> 31) & jnp.int32(0x7FFFFFFF))
    nb, nt = s.shape

    mx = jnp.max(s, axis=1, keepdims=True)
    z = jnp.sum(jnp.exp(s - mx), axis=1, keepdims=True)

    MIN32 = jnp.int32(-2**31)
    kk = jnp.int32(k_rank)

    def bs_body(i, carry):
        p_u, c_less = carry
        bit = jnp.int32(1) << (jnp.int32(31) - i)
        q_u = p_u | bit
        thr = q_u ^ MIN32
        c = jnp.sum((key < thr).astype(jnp.int32), axis=1, keepdims=True)
        accept = c < kk
        return (jnp.where(accept, q_u, p_u), jnp.where(accept, c, c_less))

    p0 = jnp.zeros((nb, 1), jnp.int32)
    p_u, c_less = lax.fori_loop(0, 32, bs_body, (p0, p0))
    theta = p_u ^ MIN32

    is_tie = key == theta
    r = kk - c_less
    idx = lax.broadcasted_iota(jnp.int32, s.shape, 1)

    def bs2_body(i, p2):
        bit = jnp.int32(1) << (jnp.int32(12) - i)
        q = p2 | bit
        c = jnp.sum((is_tie & (idx < q)).astype(jnp.int32), axis=1, keepdims=True)
        return jnp.where(c < r, q, p2)

    cut = lax.fori_loop(0, 13, bs2_body, jnp.zeros((nb, 1), jnp.int32))

    masked = (key < theta) | (is_tie & (idx <= cut))
    maw_ref[...] = jnp.where(masked, 0.0, jnp.exp(s - mx) / z)


def _apply_body(x_ref, m_ref, h0_ref, o_ref):
    maw = m_ref[...]
    o_ref[...] = x_ref[...] * maw + (1.0 - maw) * h0_ref[...]


def kernel(x, W, b, v_w, v_b, H0):
    nb, nt, nd = x.shape
    k_rank = int(nt * 0.7)
    x2 = x.reshape(nb * nt, nd)
    wt = W.T

    tba = 2048
    nblk = nb * nt // tba

    scores = pl.pallas_call(
        _scores_body,
        grid=(nblk,),
        in_specs=[
            pl.BlockSpec((tba, nd), lambda i: (i, 0)),
            pl.BlockSpec((nd, nd), lambda i: (0, 0)),
            pl.BlockSpec((1, nd), lambda i: (0, 0)),
            pl.BlockSpec((nd, 1), lambda i: (0, 0)),
            pl.BlockSpec((1, 1), lambda i: (0, 0)),
        ],
        out_specs=pl.BlockSpec((tba, 1), lambda i: (i, 0)),
        out_shape=jax.ShapeDtypeStruct((nb * nt, 1), jnp.float32),
        compiler_params=pltpu.CompilerParams(
            dimension_semantics=("arbitrary",)),
    )(x2, wt, b.reshape(1, nd), v_w.reshape(nd, 1), v_b.reshape(1, 1))

    maw = _sc_mask(scores.reshape(nb, nt), k_rank).reshape(nb, nt)

    tbc = 2048
    out = pl.pallas_call(
        _apply_body,
        grid=(nb * nt // tbc,),
        in_specs=[
            pl.BlockSpec((tbc, nd), lambda i: (i, 0)),
            pl.BlockSpec((tbc, 1), lambda i: (i, 0)),
            pl.BlockSpec((1, nd), lambda i: (0, 0)),
        ],
        out_specs=pl.BlockSpec((tbc, nd), lambda i: (i, 0)),
        out_shape=jax.ShapeDtypeStruct((nb * nt, nd), jnp.float32),
        compiler_params=pltpu.CompilerParams(
            dimension_semantics=("parallel",)),
    )(x2, maw.reshape(nb * nt, 1), H0.reshape(1, nd))

    return out.reshape(nb, nt, nd), maw


# speedup vs baseline: 1.6662x; 1.0016x over previous
import functools

import jax
import jax.numpy as jnp
from jax import lax
from jax.experimental import pallas as pl
from jax.experimental.pallas import tpu as pltpu
from jax.experimental.pallas import tpu_sc as plsc


def _scores_body(x_ref, wt_ref, b_ref, vw_ref, vb_ref, s_ref):
    xb = x_ref[...]
    h = jnp.tanh(
        lax.dot_general(xb, wt_ref[...], (((1,), (0,)), ((), ())),
                        preferred_element_type=jnp.float32,
                        precision=lax.Precision.DEFAULT)
        + b_ref[...])
    s = lax.dot_general(h, vw_ref[...], (((1,), (0,)), ((), ())),
                        preferred_element_type=jnp.float32,
                        precision=lax.Precision.DEFAULT)
    s_ref[...] = s + vb_ref[0, 0]


def _sc_mask(scores, k_rank):
    nb, nt = scores.shape
    npr = nt // 16
    nchunk = npr // 16
    kk = jnp.int32(k_rank)
    MIN32 = jnp.int32(-2**31)

    def body(s_hbm, maw_hbm, s_v, k_v, o_v, bf_f, bf_i, all_f, all_i, h2_v,
             sh_f, sh_i):
        cid = lax.axis_index("c")
        wid = lax.axis_index("s")
        iota = lax.iota(jnp.int32, 16)

        def xmax_f(v):
            for st in (8, 4, 2, 1):
                bf_f[...] = v
                v = jnp.maximum(v, plsc.load_gather(bf_f, [iota ^ st]))
            return v

        def xsum_f(v):
            for st in (8, 4, 2, 1):
                bf_f[...] = v
                v = v + plsc.load_gather(bf_f, [iota ^ st])
            return v

        def prefix_i(v):
            for st in (1, 2, 4, 8):
                bf_i[...] = v
                g = plsc.load_gather(bf_i, [jnp.maximum(iota - st, 0)])
                v = v + jnp.where(iota >= st, g, jnp.int32(0))
            return v

        def xchg_f(vec, buf):
            bf_f[...] = vec
            pltpu.sync_copy(bf_f, sh_f.at[pl.ds(buf * 256 + wid * 16, 16)])
            plsc.subcore_barrier()
            pltpu.sync_copy(sh_f.at[pl.ds(buf * 256, 256)], all_f)
            return plsc.load_gather(all_f, [iota * 16])

        def row_body(rr, carry):
            row = cid * (nb // 2) + rr
            base = row * nt + wid * npr
            pltpu.sync_copy(s_hbm.at[pl.ds(base, npr)], s_v)

            def sw1(i, mx):
                s = s_v[pl.ds(i * 16, 16)]
                s = jnp.where(s == 0.0, 0.0, s)
                bits = lax.bitcast_convert_type(s, jnp.int32)
                k_v[pl.ds(i * 16, 16)] = bits ^ ((bits >> 31) | MIN32)
                return jnp.maximum(mx, s)

            mxv = lax.fori_loop(
                0, nchunk, sw1, jnp.full((16,), -jnp.inf, jnp.float32))
            gmax = xmax_f(xchg_f(xmax_f(mxv), 0))

            def sw2(i, acc):
                e = jnp.exp(s_v[pl.ds(i * 16, 16)] - gmax)
                o_v[pl.ds(i * 16, 16)] = e
                return acc + e

            accv = lax.fori_loop(
                0, nchunk, sw2, jnp.zeros((16,), jnp.float32))
            z = xsum_f(xchg_f(xsum_f(accv), 1))
            inv_z = 1.0 / z

            zi = jnp.zeros((16,), jnp.int32)
            ones = jnp.full((16,), 1, jnp.int32)

            def rnd(t, st, final=False):
                p_u, hi_mask, rcur, before = st
                shift = zi + (jnp.int32(28) - 4 * t)

                def zb(l, c):
                    h2_v[pl.ds(l * 16, 16)] = zi
                    return c
                lax.fori_loop(0, 16, zb, 0)

                def hsweep(i, c):
                    ku = k_v[pl.ds(i * 16, 16)]
                    nib = lax.shift_right_logical(ku, shift) & jnp.int32(0xF)
                    match = (ku & hi_mask) == p_u
                    plsc.addupdate_scatter(
                        h2_v, [nib * 16 + iota], ones, mask=match)
                    return c
                lax.fori_loop(0, nchunk, hsweep, 0)

                lbin = zi
                for l in range(16):
                    lbin = lbin + plsc.load_gather(h2_v, [iota * 16 + l])

                bf_i[...] = lbin
                pltpu.sync_copy(
                    bf_i, sh_i.at[pl.ds((t % 2) * 256 + wid * 16, 16)])
                plsc.subcore_barrier()
                pltpu.sync_copy(sh_i.at[pl.ds((t % 2) * 256, 256)], all_i)
                gbin = zi
                for l in range(16):
                    gbin = gbin + all_i[pl.ds(l * 16, 16)]

                cum = prefix_i(gbin)
                chosen = plsc.all_reduce_population_count(cum < rcur)
                bf_i[...] = cum - gbin
                sel_excl = plsc.load_gather(bf_i, [chosen])
                p_u = p_u | lax.shift_left(chosen, shift)
                hi_mask = hi_mask | lax.shift_left(zi + 0xF, shift)
                if final:
                    pcnt = plsc.load_gather(all_i, [iota * 16 + chosen])
                    excl_v = prefix_i(pcnt) - pcnt
                    bf_i[...] = excl_v
                    before = plsc.load_gather(bf_i, [zi + wid])
                return (p_u, hi_mask, rcur - sel_excl, before)

            st7 = lax.fori_loop(0, 7, rnd, (zi, zi, zi + kk, zi))
            theta, _, r, before = rnd(7, st7, final=True)
            rem = r - before

            theta_s = theta ^ MIN32

            def sw3(i, run):
                kv = k_v[pl.ds(i * 16, 16)]
                tie = kv == theta
                pp = prefix_i(jnp.where(tie, jnp.int32(1), jnp.int32(0)))
                sel = tie & ((run + pp) <= rem)
                masked = ((kv ^ MIN32) < theta_s) | sel
                o_v[pl.ds(i * 16, 16)] = jnp.where(
                    masked, 0.0, o_v[pl.ds(i * 16, 16)] * inv_z)
                return run + plsc.all_reduce_population_count(tie)

            lax.fori_loop(0, nchunk, sw3, zi)
            pltpu.sync_copy(o_v, maw_hbm.at[pl.ds(base, npr)])
            return carry

        lax.fori_loop(0, nb // 2, row_body, jnp.int32(0))

    return pl.kernel(
        body,
        out_type=jax.ShapeDtypeStruct((nb * nt,), jnp.float32),
        mesh=plsc.VectorSubcoreMesh(core_axis_name="c", subcore_axis_name="s"),
        compiler_params=pltpu.CompilerParams(needs_layout_passes=False),
        scratch_types=[
            pltpu.VMEM((npr,), jnp.float32),
            pltpu.VMEM((npr,), jnp.int32),
            pltpu.VMEM((npr,), jnp.float32),
            pltpu.VMEM((16,), jnp.float32),
            pltpu.VMEM((16,), jnp.int32),
            pltpu.VMEM((256,), jnp.float32),
            pltpu.VMEM((256,), jnp.int32),
            pltpu.VMEM((256,), jnp.int32),
            pltpu.VMEM_SHARED((512,), jnp.float32),
            pltpu.VMEM_SHARED((512,), jnp.int32),
        ],
    )(scores.reshape(nb * nt))


def _apply_body(x_ref, m_ref, h0_ref, o_ref):
    maw = m_ref[...]
    o_ref[...] = x_ref[...] * maw + (1.0 - maw) * h0_ref[...]


def kernel(x, W, b, v_w, v_b, H0):
    nb, nt, nd = x.shape
    k_rank = int(nt * 0.7)
    x2 = x.reshape(nb * nt, nd)
    wt = W.T

    tba = 2048
    nblk = nb * nt // tba

    scores = pl.pallas_call(
        _scores_body,
        grid=(nblk,),
        in_specs=[
            pl.BlockSpec((tba, nd), lambda i: (i, 0)),
            pl.BlockSpec((nd, nd), lambda i: (0, 0)),
            pl.BlockSpec((1, nd), lambda i: (0, 0)),
            pl.BlockSpec((nd, 1), lambda i: (0, 0)),
            pl.BlockSpec((1, 1), lambda i: (0, 0)),
        ],
        out_specs=pl.BlockSpec((tba, 1), lambda i: (i, 0)),
        out_shape=jax.ShapeDtypeStruct((nb * nt, 1), jnp.float32),
        compiler_params=pltpu.CompilerParams(
            dimension_semantics=("arbitrary",)),
    )(x2, wt, b.reshape(1, nd), v_w.reshape(nd, 1), v_b.reshape(1, 1))

    maw = _sc_mask(scores.reshape(nb, nt), k_rank).reshape(nb, nt)

    tbc = 2048
    out = pl.pallas_call(
        _apply_body,
        grid=(nb * nt // tbc,),
        in_specs=[
            pl.BlockSpec((tbc, nd), lambda i: (i, 0)),
            pl.BlockSpec((tbc, 1), lambda i: (i, 0)),
            pl.BlockSpec((1, nd), lambda i: (0, 0)),
        ],
        out_specs=pl.BlockSpec((tbc, nd), lambda i: (i, 0)),
        out_shape=jax.ShapeDtypeStruct((nb * nt, nd), jnp.float32),
        compiler_params=pltpu.CompilerParams(
            dimension_semantics=("parallel",)),
    )(x2, maw.reshape(nb * nt, 1), H0.reshape(1, nd))

    return out.reshape(nb, nt, nd), maw
